# baseline (speedup 1.0000x reference)
import functools

import jax
import jax.numpy as jnp
from jax import lax
from jax.experimental import pallas as pl
from jax.experimental.pallas import tpu as pltpu
from jax.experimental.pallas import tpu_sc as plsc

_N, _E, _G = 10000, 160000, 512
_NP, _EP, _GP = 10240, 163840, 520
_NC, _NS = 2, 16
_NW = _NC * _NS
_CH = 128
_EC = _EP // _CH
_ECW = _EC // _NW
_KG = 20

_F32 = jnp.float32
_BE = 640


def _sc_mesh():
    return plsc.VectorSubcoreMesh(
        core_axis_name="c", subcore_axis_name="s",
        num_cores=_NC, num_subcores=_NS)


_SC_PARAMS = pltpu.CompilerParams(use_tc_tiling_on_sc=False)


def _proj_h0_body(nf_ref, wp_ref, bp_ref, o_ref):
    o_ref[...] = jax.nn.relu(
        jnp.dot(nf_ref[...], wp_ref[...], preferred_element_type=_F32)
        + bp_ref[...])


def _proj_h0(nf_p, wp_p, bp_r):
    return pl.pallas_call(
        _proj_h0_body,
        grid=(_NP // 2048,),
        in_specs=[
            pl.BlockSpec((2048, 136), lambda i: (i, 0)),
            pl.BlockSpec((136, 32), lambda i: (0, 0)),
            pl.BlockSpec((1, 32), lambda i: (0, 0)),
        ],
        out_specs=pl.BlockSpec((2048, 32), lambda i: (i, 0)),
        out_shape=jax.ShapeDtypeStruct((_NP, 32), _F32),
    )(nf_p, wp_p, bp_r)


def _edge_pre_body(eft_ref, ef_ref, w1t_ref, b1c_ref, wpe_ref, bpe_ref,
                   eht_ref, eemb_ref):
    eht_ref[...] = jax.nn.relu(
        jnp.dot(w1t_ref[...], eft_ref[...], preferred_element_type=_F32)
        + b1c_ref[...])
    eemb = jax.nn.relu(
        jnp.dot(ef_ref[...], wpe_ref[...], preferred_element_type=_F32)
        + bpe_ref[...])
    eemb_ref[...] = eemb.reshape(_BE // _CH, _CH, 64)


def _edge_pre(eft_p, ef_p, w1t_p, b1c, wpe_p, bpe_r):
    return pl.pallas_call(
        _edge_pre_body,
        grid=(_EP // _BE,),
        in_specs=[
            pl.BlockSpec((8, _BE), lambda i: (0, i)),
            pl.BlockSpec((_BE, 8), lambda i: (i, 0)),
            pl.BlockSpec((64, 8), lambda i: (0, 0)),
            pl.BlockSpec((64, 1), lambda i: (0, 0)),
            pl.BlockSpec((8, 64), lambda i: (0, 0)),
            pl.BlockSpec((1, 64), lambda i: (0, 0)),
        ],
        out_specs=[
            pl.BlockSpec((64, _BE), lambda i: (i, 0)),
            pl.BlockSpec((_BE // _CH, _CH, 64), lambda i: (i, 0, 0)),
        ],
        out_shape=[
            jax.ShapeDtypeStruct(((_EP // _BE) * 64, _BE), _F32),
            jax.ShapeDtypeStruct((_EC, _CH, 64), _F32),
        ],
    )(eft_p, ef_p, w1t_p, b1c, wpe_p, bpe_r)


def _msg_body(eht_ref, hs_ref, w2t_ref, be2t_ref, msg_ref):
    ht = hs_ref[...].reshape(_BE, 32).T
    u = (eht_ref[...][:, None, :] * ht[None, :, :]).reshape(64 * 32, _BE)
    msgt = jnp.dot(w2t_ref[...], u, preferred_element_type=_F32)
    msgt = msgt + jnp.dot(be2t_ref[...], ht, preferred_element_type=_F32)
    msg_ref[...] = msgt.T.reshape(_BE // _CH, _CH, 32)


def _msg(eht, hs, w2t, be2t):
    return pl.pallas_call(
        _msg_body,
        grid=(_EP // _BE,),
        in_specs=[
            pl.BlockSpec((64, _BE), lambda i: (i, 0)),
            pl.BlockSpec((_BE // _CH, _CH, 32), lambda i: (i, 0, 0)),
            pl.BlockSpec((32, 2048), lambda i: (0, 0)),
            pl.BlockSpec((32, 32), lambda i: (0, 0)),
        ],
        out_specs=pl.BlockSpec((_BE // _CH, _CH, 32), lambda i: (i, 0, 0)),
        out_shape=jax.ShapeDtypeStruct((_EC, _CH, 32), _F32),
    )(eht, hs, w2t, be2t)


def _gru_body(parts_ref, h_ref, hid_ref, wih_ref, whh_ref, bih_ref, bhh_ref,
              bconv_ref, ho_ref, hido_ref):
    hid = hid_ref[...]
    m = jax.nn.relu(parts_ref[0] + parts_ref[1] + bconv_ref[...])
    gi = jnp.dot(m, wih_ref[...], preferred_element_type=_F32) + bih_ref[...]
    gh = jnp.dot(hid, whh_ref[...], preferred_element_type=_F32) + bhh_ref[...]
    r = jax.nn.sigmoid(gi[:, 0:32] + gh[:, 0:32])
    z = jax.nn.sigmoid(gi[:, 32:64] + gh[:, 32:64])
    n = jnp.tanh(gi[:, 64:96] + r * gh[:, 64:96])
    gru = (1.0 - z) * n + z * hid
    ho_ref[...] = gru + h_ref[...]
    hido_ref[...] = gru


def _gru(parts, h, hid, wiht, whht, bih_r, bhh_r, bconv_r):
    return pl.pallas_call(
        _gru_body,
        grid=(_NP // 2048,),
        in_specs=[
            pl.BlockSpec((2, 2048, 32), lambda i: (0, i, 0)),
            pl.BlockSpec((2048, 32), lambda i: (i, 0)),
            pl.BlockSpec((2048, 32), lambda i: (i, 0)),
            pl.BlockSpec((32, 96), lambda i: (0, 0)),
            pl.BlockSpec((32, 96), lambda i: (0, 0)),
            pl.BlockSpec((1, 96), lambda i: (0, 0)),
            pl.BlockSpec((1, 96), lambda i: (0, 0)),
            pl.BlockSpec((1, 32), lambda i: (0, 0)),
        ],
        out_specs=[
            pl.BlockSpec((2048, 32), lambda i: (i, 0)),
            pl.BlockSpec((2048, 32), lambda i: (i, 0)),
        ],
        out_shape=[
            jax.ShapeDtypeStruct((_NP, 32), _F32),
            jax.ShapeDtypeStruct((_NP, 32), _F32),
        ],
    )(parts, h, hid, wiht, whht, bih_r, bhh_r, bconv_r)


def _bn(x, gamma, beta):
    mu = jnp.mean(x, axis=0, keepdims=True)
    var = jnp.mean((x - mu) ** 2, axis=0, keepdims=True)
    return gamma * (x - mu) * jax.lax.rsqrt(var + 1e-5) + beta


def _head_body(ph_ref, pe_ref, w1_ref, b1_ref, g1_ref, bt1_ref,
               w2_ref, b2_ref, g2_ref, bt2_ref, w3_ref, b3_ref, o_ref):
    mol = jnp.concatenate(
        [ph_ref[0] + ph_ref[1], pe_ref[0] + pe_ref[1]], axis=1)
    mol = mol - jnp.max(mol, axis=1, keepdims=True)
    e = jnp.exp(mol)
    p = e / jnp.sum(e, axis=1, keepdims=True)
    x = jnp.dot(p, w1_ref[...], preferred_element_type=_F32) + b1_ref[...]
    x = jax.nn.relu(_bn(x, g1_ref[...], bt1_ref[...]))
    x = jnp.dot(x, w2_ref[...], preferred_element_type=_F32) + b2_ref[...]
    x = jax.nn.relu(_bn(x, g2_ref[...], bt2_ref[...]))
    o_ref[...] = jnp.dot(x, w3_ref[...], preferred_element_type=_F32) + b3_ref[...]


def _head(ph, pe, w1, b1, g1, bt1, w2, b2, g2, bt2, w3, b3):
    full = lambda s: pl.BlockSpec(s, lambda i, _s=s: tuple(0 for _ in _s))
    return pl.pallas_call(
        _head_body,
        grid=(1,),
        in_specs=[
            full((2, _G, 32)), full((2, _G, 64)),
            full((96, 300)), full((1, 300)), full((1, 300)), full((1, 300)),
            full((300, 256)), full((1, 256)), full((1, 256)), full((1, 256)),
            full((256, 138)), full((1, 138)),
        ],
        out_specs=full((_G, 138)),
        out_shape=jax.ShapeDtypeStruct((_G, 138), _F32),
    )(ph, pe, w1, b1, g1, bt1, w2, b2, g2, bt2, w3, b3)


_KB = 10


def _gather_body(h_hbm, srcs_hbm, out_hbm, idx_v, rows_v, sem, wsem, htab):
    c = lax.axis_index("c")
    s = lax.axis_index("s")
    w = s * _NC + c
    rpt = _NP // _NS
    pltpu.sync_copy(h_hbm.at[pl.ds(s * rpt, rpt)], htab.at[pl.ds(s * rpt, rpt)])
    pltpu.sync_copy(srcs_hbm.at[pl.ds(w * _ECW, _ECW)], idx_v)
    plsc.subcore_barrier()
    wr = [None, None]
    for r in range(_ECW // _KB):
        b = r % 2
        if wr[b] is not None:
            wr[b].wait()
        cps = [
            pltpu.async_copy(htab.at[idx_v.at[r * _KB + j]],
                             rows_v.at[b, j], sem)
            for j in range(_KB)
        ]
        for cp in cps:
            cp.wait()
        wr[b] = pltpu.async_copy(
            rows_v.at[b], out_hbm.at[pl.ds(w * _ECW + r * _KB, _KB)], wsem)
    for x in wr:
        x.wait()


def _sc_gather(h, srcs):
    k = functools.partial(
        pl.kernel,
        out_type=jax.ShapeDtypeStruct((_EC, _CH, 32), _F32),
        mesh=_sc_mesh(),
        compiler_params=_SC_PARAMS,
        scratch_types=[
            pltpu.VMEM((_ECW, _CH), jnp.int32),
            pltpu.VMEM((2, _KB, _CH, 32), _F32),
            pltpu.SemaphoreType.DMA,
            pltpu.SemaphoreType.DMA,
            pltpu.VMEM_SHARED((_NP, 32), _F32),
        ],
    )(_gather_body)
    return k(h, srcs)


def _scatter_body(msg3_hbm, dsts_hbm, z32_hbm, parts_hbm,
                  idx_v, rows_v, sem, acc_sh):
    c = lax.axis_index("c")
    s = lax.axis_index("s")
    w = s * _NC + c
    rpt = _NP // _NS
    pltpu.sync_copy(z32_hbm.at[pl.ds(s * rpt, rpt)],
                    acc_sh.at[pl.ds(s * rpt, rpt)])
    pltpu.sync_copy(dsts_hbm.at[pl.ds(w * _ECW, _ECW)], idx_v)
    plsc.subcore_barrier()
    for half in range(_ECW // _KG):
        pltpu.sync_copy(msg3_hbm.at[pl.ds(w * _ECW + half * _KG, _KG)],
                        rows_v)
        cps = [
            pltpu.async_copy(rows_v.at[j],
                             acc_sh.at[idx_v.at[half * _KG + j]],
                             sem, add=True)
            for j in range(_KG)
        ]
        for cp in cps:
            cp.wait()
    plsc.subcore_barrier()
    pltpu.sync_copy(acc_sh.at[pl.ds(s * rpt, rpt)],
                    parts_hbm.at[c, pl.ds(s * rpt, rpt)])


def _sc_scatter(msg3, dsts, z32):
    k = functools.partial(
        pl.kernel,
        out_type=jax.ShapeDtypeStruct((_NC, _NP, 32), _F32),
        mesh=_sc_mesh(),
        compiler_params=_SC_PARAMS,
        scratch_types=[
            pltpu.VMEM((_ECW, _CH), jnp.int32),
            pltpu.VMEM((_KG, _CH, 32), _F32),
            pltpu.SemaphoreType.DMA,
            pltpu.VMEM_SHARED((_NP, 32), _F32),
        ],
    )(_scatter_body)
    return k(msg3, dsts, z32)


_KR = 4


def _readout_body(h_hbm, eemb3_hbm, srcs_hbm, dsts_hbm, n2g_hbm,
                  z32_hbm, z64_hbm, zg32_hbm, zg64_hbm,
                  outh_hbm, oute_hbm,
                  idxs_v, idxd_v, idxn_v, rows32_v, rows64_v, sem,
                  acc_h, acc_e, acc_gh, acc_ge):
    c = lax.axis_index("c")
    s = lax.axis_index("s")
    w = s * _NC + c
    rpt = _NP // _NS
    gpt = _G // _NS
    npt = (_NP // _CH) // _NS
    pltpu.sync_copy(z32_hbm.at[pl.ds(s * rpt, rpt)],
                    acc_h.at[pl.ds(s * rpt, rpt)])
    pltpu.sync_copy(z64_hbm.at[pl.ds(s * rpt, rpt)],
                    acc_e.at[pl.ds(s * rpt, rpt)])
    pltpu.sync_copy(zg32_hbm.at[pl.ds(s * gpt, gpt)],
                    acc_gh.at[pl.ds(s * gpt, gpt)])
    pltpu.sync_copy(zg64_hbm.at[pl.ds(s * gpt, gpt)],
                    acc_ge.at[pl.ds(s * gpt, gpt)])
    @pl.when(s == 0)
    def _():
        pltpu.sync_copy(zg32_hbm.at[pl.ds(_G, _GP - _G)],
                        acc_gh.at[pl.ds(_G, _GP - _G)])
        pltpu.sync_copy(zg64_hbm.at[pl.ds(_G, _GP - _G)],
                        acc_ge.at[pl.ds(_G, _GP - _G)])
    pltpu.sync_copy(srcs_hbm.at[pl.ds(w * _ECW, _ECW)], idxs_v)
    pltpu.sync_copy(dsts_hbm.at[pl.ds(w * _ECW, _ECW)], idxd_v)
    plsc.subcore_barrier()
    for r in range(_ECW // _KR):
        cps = [
            pltpu.async_copy(h_hbm.at[idxs_v.at[r * _KR + j]],
                             rows32_v.at[j], sem)
            for j in range(_KR)
        ]
        pltpu.sync_copy(eemb3_hbm.at[pl.ds(w * _ECW + r * _KR, _KR)],
                        rows64_v)
        for cp in cps:
            cp.wait()
        cps = [
            pltpu.async_copy(rows32_v.at[j],
                             acc_h.at[idxd_v.at[r * _KR + j]],
                             sem, add=True)
            for j in range(_KR)
        ] + [
            pltpu.async_copy(rows64_v.at[j],
                             acc_e.at[idxd_v.at[r * _KR + j]],
                             sem, add=True)
            for j in range(_KR)
        ]
        for cp in cps:
            cp.wait()
    plsc.subcore_barrier()
    pltpu.sync_copy(n2g_hbm.at[pl.ds(s * npt, npt)], idxn_v)
    for t in range(npt):
        p = s * npt + t
        pltpu.sync_copy(acc_h.at[pl.ds(p * _CH, _CH)], rows32_v.at[0])
        pltpu.sync_copy(acc_e.at[pl.ds(p * _CH, _CH)], rows64_v.at[0])
        pltpu.sync_copy(rows32_v.at[0], acc_gh.at[idxn_v.at[t]], add=True)
        pltpu.sync_copy(rows64_v.at[0], acc_ge.at[idxn_v.at[t]], add=True)
    plsc.subcore_barrier()
    pltpu.sync_copy(acc_gh.at[pl.ds(s * gpt, gpt)],
                    outh_hbm.at[c, pl.ds(s * gpt, gpt)])
    pltpu.sync_copy(acc_ge.at[pl.ds(s * gpt, gpt)],
                    oute_hbm.at[c, pl.ds(s * gpt, gpt)])


def _sc_readout(h, eemb3, srcs, dsts, n2g, z32, z64, zg32, zg64):
    k = functools.partial(
        pl.kernel,
        out_type=(
            jax.ShapeDtypeStruct((_NC, _G, 32), _F32),
            jax.ShapeDtypeStruct((_NC, _G, 64), _F32),
        ),
        mesh=_sc_mesh(),
        compiler_params=_SC_PARAMS,
        scratch_types=[
            pltpu.VMEM((_ECW, _CH), jnp.int32),
            pltpu.VMEM((_ECW, _CH), jnp.int32),
            pltpu.VMEM(((_NP // _CH) // _NS, _CH), jnp.int32),
            pltpu.VMEM((_KR, _CH, 32), _F32),
            pltpu.VMEM((_KR, _CH, 64), _F32),
            pltpu.SemaphoreType.DMA,
            pltpu.VMEM_SHARED((_NP, 32), _F32),
            pltpu.VMEM_SHARED((_NP, 64), _F32),
            pltpu.VMEM_SHARED((_GP, 32), _F32),
            pltpu.VMEM_SHARED((_GP, 64), _F32),
        ],
    )(_readout_body)
    return k(h, eemb3, srcs, dsts, n2g, z32, z64, zg32, zg64)


def kernel(node_feats, edge_feats, edge_index, node2graph, Wp, bp, We1, be1,
           We2, be2, bconv, Wih, Whh, bih, bhh, Wpe, bpe, W1, b1, g1, beta1,
           W2, b2, g2, beta2, W3, b3):
    i32 = jnp.int32
    src = edge_index[0]
    dst = edge_index[1]
    srcs = jnp.concatenate(
        [src, jnp.zeros((_EP - _E,), i32)]).reshape(_EC, _CH)
    dsts = jnp.concatenate(
        [dst, jnp.full((_EP - _E,), _N, i32)]).reshape(_EC, _CH)
    n2g = jnp.concatenate(
        [node2graph, jnp.full((_NP - _N,), _G, i32)]).reshape(_NP // _CH, _CH)
    nf_p = jnp.pad(node_feats, ((0, _NP - _N), (0, 2)))
    ef_p = jnp.pad(edge_feats, ((0, _EP - _E), (0, 2)))
    eft_p = jnp.pad(edge_feats.T, ((0, 2), (0, _EP - _E)))
    wp_p = jnp.pad(Wp, ((0, 2), (0, 0)))
    w1t_p = jnp.pad(We1.T, ((0, 0), (0, 2)))
    wpe_p = jnp.pad(Wpe, ((0, 2), (0, 0)))
    w2t = We2.reshape(64, 32, 32).reshape(64 * 32, 32).T
    be2t = be2.reshape(32, 32).T
    wiht = Wih.T
    whht = Whh.T
    z32 = jnp.zeros((_NP, 32), _F32)
    z64 = jnp.zeros((_NP, 64), _F32)
    zg32 = jnp.zeros((_GP, 32), _F32)
    zg64 = jnp.zeros((_GP, 64), _F32)

    h = _proj_h0(nf_p, wp_p, bp.reshape(1, 32))
    eht, eemb3 = _edge_pre(eft_p, ef_p, w1t_p, be1.reshape(64, 1),
                           wpe_p, bpe.reshape(1, 64))
    hidden = h
    for _ in range(3):
        hs3 = _sc_gather(h, srcs)
        msg3 = _msg(eht, hs3, w2t, be2t)
        parts = _sc_scatter(msg3, dsts, z32)
        h, hidden = _gru(parts, h, hidden, wiht, whht,
                         bih.reshape(1, 96), bhh.reshape(1, 96),
                         bconv.reshape(1, 32))
    ph, pe = _sc_readout(h, eemb3, srcs, dsts, n2g,
                         z32, z64, zg32, zg64)
    return _head(ph, pe, W1, b1.reshape(1, 300), g1.reshape(1, 300),
                 beta1.reshape(1, 300), W2, b2.reshape(1, 256),
                 g2.reshape(1, 256), beta2.reshape(1, 256),
                 W3, b3.reshape(1, 138))


# speedup vs baseline: 1.0005x; 1.0005x over previous
import functools

import jax
import jax.numpy as jnp
from jax import lax
from jax.experimental import pallas as pl
from jax.experimental.pallas import tpu as pltpu
from jax.experimental.pallas import tpu_sc as plsc

_N, _E, _G = 10000, 160000, 512
_NP, _EP, _GP = 10240, 163840, 520
_NC, _NS = 2, 16
_NW = _NC * _NS
_CH = 128
_EC = _EP // _CH
_ECW = _EC // _NW
_KG = 20

_F32 = jnp.float32
_BE = 640


def _sc_mesh():
    return plsc.VectorSubcoreMesh(
        core_axis_name="c", subcore_axis_name="s",
        num_cores=_NC, num_subcores=_NS)


_SC_PARAMS = pltpu.CompilerParams(use_tc_tiling_on_sc=False)


def _proj_h0_body(nf_ref, wp_ref, bp_ref, o_ref):
    o_ref[...] = jax.nn.relu(
        jnp.dot(nf_ref[...], wp_ref[...], preferred_element_type=_F32)
        + bp_ref[...])


def _proj_h0(nf_p, wp_p, bp_r):
    return pl.pallas_call(
        _proj_h0_body,
        grid=(_NP // 2048,),
        in_specs=[
            pl.BlockSpec((2048, 136), lambda i: (i, 0)),
            pl.BlockSpec((136, 32), lambda i: (0, 0)),
            pl.BlockSpec((1, 32), lambda i: (0, 0)),
        ],
        out_specs=pl.BlockSpec((2048, 32), lambda i: (i, 0)),
        out_shape=jax.ShapeDtypeStruct((_NP, 32), _F32),
    )(nf_p, wp_p, bp_r)


def _edge_pre_body(eft_ref, ef_ref, w1t_ref, b1c_ref, wpe_ref, bpe_ref,
                   eht_ref, eemb_ref):
    eht_ref[...] = jax.nn.relu(
        jnp.dot(w1t_ref[...], eft_ref[...], preferred_element_type=_F32)
        + b1c_ref[...])
    eemb_ref[...] = jax.nn.relu(
        jnp.dot(ef_ref[...], wpe_ref[...], preferred_element_type=_F32)
        + bpe_ref[...])


def _edge_pre(eft_p, ef_p, w1t_p, b1c, wpe_p, bpe_r):
    return pl.pallas_call(
        _edge_pre_body,
        grid=(_EP // _BE,),
        in_specs=[
            pl.BlockSpec((8, _BE), lambda i: (0, i)),
            pl.BlockSpec((_BE, 8), lambda i: (i, 0)),
            pl.BlockSpec((64, 8), lambda i: (0, 0)),
            pl.BlockSpec((64, 1), lambda i: (0, 0)),
            pl.BlockSpec((8, 64), lambda i: (0, 0)),
            pl.BlockSpec((1, 64), lambda i: (0, 0)),
        ],
        out_specs=[
            pl.BlockSpec((64, _BE), lambda i: (i, 0)),
            pl.BlockSpec((_BE, 64), lambda i: (i, 0)),
        ],
        out_shape=[
            jax.ShapeDtypeStruct(((_EP // _BE) * 64, _BE), _F32),
            jax.ShapeDtypeStruct((_EP, 64), _F32),
        ],
    )(eft_p, ef_p, w1t_p, b1c, wpe_p, bpe_r)


def _msg_body(eht_ref, hs_ref, w2t_ref, be2t_ref, msg_ref):
    ht = hs_ref[...].T
    u = (eht_ref[...][:, None, :] * ht[None, :, :]).reshape(64 * 32, _BE)
    msgt = jnp.dot(w2t_ref[...], u, preferred_element_type=_F32)
    msgt = msgt + jnp.dot(be2t_ref[...], ht, preferred_element_type=_F32)
    msg_ref[...] = msgt.T


def _msg(eht, hs, w2t, be2t):
    return pl.pallas_call(
        _msg_body,
        grid=(_EP // _BE,),
        in_specs=[
            pl.BlockSpec((64, _BE), lambda i: (i, 0)),
            pl.BlockSpec((_BE, 32), lambda i: (i, 0)),
            pl.BlockSpec((32, 2048), lambda i: (0, 0)),
            pl.BlockSpec((32, 32), lambda i: (0, 0)),
        ],
        out_specs=pl.BlockSpec((_BE, 32), lambda i: (i, 0)),
        out_shape=jax.ShapeDtypeStruct((_EP, 32), _F32),
    )(eht, hs, w2t, be2t)


def _gru_body(parts_ref, h_ref, hid_ref, wih_ref, whh_ref, bih_ref, bhh_ref,
              bconv_ref, ho_ref, hido_ref):
    hid = hid_ref[...]
    m = jax.nn.relu(parts_ref[0] + parts_ref[1] + bconv_ref[...])
    gi = jnp.dot(m, wih_ref[...], preferred_element_type=_F32) + bih_ref[...]
    gh = jnp.dot(hid, whh_ref[...], preferred_element_type=_F32) + bhh_ref[...]
    r = jax.nn.sigmoid(gi[:, 0:32] + gh[:, 0:32])
    z = jax.nn.sigmoid(gi[:, 32:64] + gh[:, 32:64])
    n = jnp.tanh(gi[:, 64:96] + r * gh[:, 64:96])
    gru = (1.0 - z) * n + z * hid
    ho_ref[...] = gru + h_ref[...]
    hido_ref[...] = gru


def _gru(parts, h, hid, wiht, whht, bih_r, bhh_r, bconv_r):
    return pl.pallas_call(
        _gru_body,
        grid=(_NP // 2048,),
        in_specs=[
            pl.BlockSpec((2, 2048, 32), lambda i: (0, i, 0)),
            pl.BlockSpec((2048, 32), lambda i: (i, 0)),
            pl.BlockSpec((2048, 32), lambda i: (i, 0)),
            pl.BlockSpec((32, 96), lambda i: (0, 0)),
            pl.BlockSpec((32, 96), lambda i: (0, 0)),
            pl.BlockSpec((1, 96), lambda i: (0, 0)),
            pl.BlockSpec((1, 96), lambda i: (0, 0)),
            pl.BlockSpec((1, 32), lambda i: (0, 0)),
        ],
        out_specs=[
            pl.BlockSpec((2048, 32), lambda i: (i, 0)),
            pl.BlockSpec((2048, 32), lambda i: (i, 0)),
        ],
        out_shape=[
            jax.ShapeDtypeStruct((_NP, 32), _F32),
            jax.ShapeDtypeStruct((_NP, 32), _F32),
        ],
    )(parts, h, hid, wiht, whht, bih_r, bhh_r, bconv_r)


def _bn(x, gamma, beta):
    mu = jnp.mean(x, axis=0, keepdims=True)
    var = jnp.mean((x - mu) ** 2, axis=0, keepdims=True)
    return gamma * (x - mu) * jax.lax.rsqrt(var + 1e-5) + beta


def _head_body(ph_ref, pe_ref, w1_ref, b1_ref, g1_ref, bt1_ref,
               w2_ref, b2_ref, g2_ref, bt2_ref, w3_ref, b3_ref, o_ref):
    mol = jnp.concatenate(
        [ph_ref[0] + ph_ref[1], pe_ref[0] + pe_ref[1]], axis=1)
    mol = mol - jnp.max(mol, axis=1, keepdims=True)
    e = jnp.exp(mol)
    p = e / jnp.sum(e, axis=1, keepdims=True)
    x = jnp.dot(p, w1_ref[...], preferred_element_type=_F32) + b1_ref[...]
    x = jax.nn.relu(_bn(x, g1_ref[...], bt1_ref[...]))
    x = jnp.dot(x, w2_ref[...], preferred_element_type=_F32) + b2_ref[...]
    x = jax.nn.relu(_bn(x, g2_ref[...], bt2_ref[...]))
    o_ref[...] = jnp.dot(x, w3_ref[...], preferred_element_type=_F32) + b3_ref[...]


def _head(ph, pe, w1, b1, g1, bt1, w2, b2, g2, bt2, w3, b3):
    full = lambda s: pl.BlockSpec(s, lambda i, _s=s: tuple(0 for _ in _s))
    return pl.pallas_call(
        _head_body,
        grid=(1,),
        in_specs=[
            full((2, _G, 32)), full((2, _G, 64)),
            full((96, 300)), full((1, 300)), full((1, 300)), full((1, 300)),
            full((300, 256)), full((1, 256)), full((1, 256)), full((1, 256)),
            full((256, 138)), full((1, 138)),
        ],
        out_specs=full((_G, 138)),
        out_shape=jax.ShapeDtypeStruct((_G, 138), _F32),
    )(ph, pe, w1, b1, g1, bt1, w2, b2, g2, bt2, w3, b3)


_KB = 10


def _gather_body(h_hbm, srcs_hbm, out_hbm, idx_v, rows_v, sem, wsem, htab):
    c = lax.axis_index("c")
    s = lax.axis_index("s")
    w = s * _NC + c
    rpt = _NP // _NS
    pltpu.sync_copy(h_hbm.at[pl.ds(s * rpt, rpt)], htab.at[pl.ds(s * rpt, rpt)])
    pltpu.sync_copy(srcs_hbm.at[pl.ds(w * _ECW, _ECW)], idx_v)
    plsc.subcore_barrier()
    wr = [None, None]
    for r in range(_ECW // _KB):
        b = r % 2
        if wr[b] is not None:
            wr[b].wait()
        cps = [
            pltpu.async_copy(htab.at[idx_v.at[r * _KB + j]],
                             rows_v.at[b, j], sem)
            for j in range(_KB)
        ]
        for cp in cps:
            cp.wait()
        wr[b] = pltpu.async_copy(
            rows_v.at[b], out_hbm.at[pl.ds(w * _ECW + r * _KB, _KB)], wsem)
    for x in wr:
        x.wait()


def _sc_gather(h, srcs):
    k = functools.partial(
        pl.kernel,
        out_type=jax.ShapeDtypeStruct((_EC, _CH, 32), _F32),
        mesh=_sc_mesh(),
        compiler_params=_SC_PARAMS,
        scratch_types=[
            pltpu.VMEM((_ECW, _CH), jnp.int32),
            pltpu.VMEM((2, _KB, _CH, 32), _F32),
            pltpu.SemaphoreType.DMA,
            pltpu.SemaphoreType.DMA,
            pltpu.VMEM_SHARED((_NP, 32), _F32),
        ],
    )(_gather_body)
    return k(h, srcs)


def _scatter_body(msg3_hbm, dsts_hbm, z32_hbm, parts_hbm,
                  idx_v, rows_v, sem, acc_sh):
    c = lax.axis_index("c")
    s = lax.axis_index("s")
    w = s * _NC + c
    rpt = _NP // _NS
    pltpu.sync_copy(z32_hbm.at[pl.ds(s * rpt, rpt)],
                    acc_sh.at[pl.ds(s * rpt, rpt)])
    pltpu.sync_copy(dsts_hbm.at[pl.ds(w * _ECW, _ECW)], idx_v)
    plsc.subcore_barrier()
    for half in range(_ECW // _KG):
        pltpu.sync_copy(msg3_hbm.at[pl.ds(w * _ECW + half * _KG, _KG)],
                        rows_v)
        cps = [
            pltpu.async_copy(rows_v.at[j],
                             acc_sh.at[idx_v.at[half * _KG + j]],
                             sem, add=True)
            for j in range(_KG)
        ]
        for cp in cps:
            cp.wait()
    plsc.subcore_barrier()
    pltpu.sync_copy(acc_sh.at[pl.ds(s * rpt, rpt)],
                    parts_hbm.at[c, pl.ds(s * rpt, rpt)])


def _sc_scatter(msg3, dsts, z32):
    k = functools.partial(
        pl.kernel,
        out_type=jax.ShapeDtypeStruct((_NC, _NP, 32), _F32),
        mesh=_sc_mesh(),
        compiler_params=_SC_PARAMS,
        scratch_types=[
            pltpu.VMEM((_ECW, _CH), jnp.int32),
            pltpu.VMEM((_KG, _CH, 32), _F32),
            pltpu.SemaphoreType.DMA,
            pltpu.VMEM_SHARED((_NP, 32), _F32),
        ],
    )(_scatter_body)
    return k(msg3, dsts, z32)


_KR = 4


def _readout_body(h_hbm, eemb3_hbm, srcs_hbm, dsts_hbm, n2g_hbm,
                  z32_hbm, z64_hbm, zg32_hbm, zg64_hbm,
                  outh_hbm, oute_hbm,
                  idxs_v, idxd_v, idxn_v, rows32_v, rows64_v, sem,
                  acc_h, acc_e, acc_gh, acc_ge):
    c = lax.axis_index("c")
    s = lax.axis_index("s")
    w = s * _NC + c
    rpt = _NP // _NS
    gpt = _G // _NS
    npt = (_NP // _CH) // _NS
    pltpu.sync_copy(z32_hbm.at[pl.ds(s * rpt, rpt)],
                    acc_h.at[pl.ds(s * rpt, rpt)])
    pltpu.sync_copy(z64_hbm.at[pl.ds(s * rpt, rpt)],
                    acc_e.at[pl.ds(s * rpt, rpt)])
    pltpu.sync_copy(zg32_hbm.at[pl.ds(s * gpt, gpt)],
                    acc_gh.at[pl.ds(s * gpt, gpt)])
    pltpu.sync_copy(zg64_hbm.at[pl.ds(s * gpt, gpt)],
                    acc_ge.at[pl.ds(s * gpt, gpt)])
    @pl.when(s == 0)
    def _():
        pltpu.sync_copy(zg32_hbm.at[pl.ds(_G, _GP - _G)],
                        acc_gh.at[pl.ds(_G, _GP - _G)])
        pltpu.sync_copy(zg64_hbm.at[pl.ds(_G, _GP - _G)],
                        acc_ge.at[pl.ds(_G, _GP - _G)])
    pltpu.sync_copy(srcs_hbm.at[pl.ds(w * _ECW, _ECW)], idxs_v)
    pltpu.sync_copy(dsts_hbm.at[pl.ds(w * _ECW, _ECW)], idxd_v)
    plsc.subcore_barrier()
    for r in range(_ECW // _KR):
        cps = [
            pltpu.async_copy(h_hbm.at[idxs_v.at[r * _KR + j]],
                             rows32_v.at[j], sem)
            for j in range(_KR)
        ]
        pltpu.sync_copy(eemb3_hbm.at[pl.ds(w * _ECW + r * _KR, _KR)],
                        rows64_v)
        for cp in cps:
            cp.wait()
        cps = [
            pltpu.async_copy(rows32_v.at[j],
                             acc_h.at[idxd_v.at[r * _KR + j]],
                             sem, add=True)
            for j in range(_KR)
        ] + [
            pltpu.async_copy(rows64_v.at[j],
                             acc_e.at[idxd_v.at[r * _KR + j]],
                             sem, add=True)
            for j in range(_KR)
        ]
        for cp in cps:
            cp.wait()
    plsc.subcore_barrier()
    pltpu.sync_copy(n2g_hbm.at[pl.ds(s * npt, npt)], idxn_v)
    for t in range(npt):
        p = s * npt + t
        pltpu.sync_copy(acc_h.at[pl.ds(p * _CH, _CH)], rows32_v.at[0])
        pltpu.sync_copy(acc_e.at[pl.ds(p * _CH, _CH)], rows64_v.at[0])
        pltpu.sync_copy(rows32_v.at[0], acc_gh.at[idxn_v.at[t]], add=True)
        pltpu.sync_copy(rows64_v.at[0], acc_ge.at[idxn_v.at[t]], add=True)
    plsc.subcore_barrier()
    pltpu.sync_copy(acc_gh.at[pl.ds(s * gpt, gpt)],
                    outh_hbm.at[c, pl.ds(s * gpt, gpt)])
    pltpu.sync_copy(acc_ge.at[pl.ds(s * gpt, gpt)],
                    oute_hbm.at[c, pl.ds(s * gpt, gpt)])


def _sc_readout(h, eemb3, srcs, dsts, n2g, z32, z64, zg32, zg64):
    k = functools.partial(
        pl.kernel,
        out_type=(
            jax.ShapeDtypeStruct((_NC, _G, 32), _F32),
            jax.ShapeDtypeStruct((_NC, _G, 64), _F32),
        ),
        mesh=_sc_mesh(),
        compiler_params=_SC_PARAMS,
        scratch_types=[
            pltpu.VMEM((_ECW, _CH), jnp.int32),
            pltpu.VMEM((_ECW, _CH), jnp.int32),
            pltpu.VMEM(((_NP // _CH) // _NS, _CH), jnp.int32),
            pltpu.VMEM((_KR, _CH, 32), _F32),
            pltpu.VMEM((_KR, _CH, 64), _F32),
            pltpu.SemaphoreType.DMA,
            pltpu.VMEM_SHARED((_NP, 32), _F32),
            pltpu.VMEM_SHARED((_NP, 64), _F32),
            pltpu.VMEM_SHARED((_GP, 32), _F32),
            pltpu.VMEM_SHARED((_GP, 64), _F32),
        ],
    )(_readout_body)
    return k(h, eemb3, srcs, dsts, n2g, z32, z64, zg32, zg64)


def kernel(node_feats, edge_feats, edge_index, node2graph, Wp, bp, We1, be1,
           We2, be2, bconv, Wih, Whh, bih, bhh, Wpe, bpe, W1, b1, g1, beta1,
           W2, b2, g2, beta2, W3, b3):
    i32 = jnp.int32
    src = edge_index[0]
    dst = edge_index[1]
    srcs = jnp.concatenate(
        [src, jnp.zeros((_EP - _E,), i32)]).reshape(_EC, _CH)
    dsts = jnp.concatenate(
        [dst, jnp.full((_EP - _E,), _N, i32)]).reshape(_EC, _CH)
    n2g = jnp.concatenate(
        [node2graph, jnp.full((_NP - _N,), _G, i32)]).reshape(_NP // _CH, _CH)
    nf_p = jnp.pad(node_feats, ((0, _NP - _N), (0, 2)))
    ef_p = jnp.pad(edge_feats, ((0, _EP - _E), (0, 2)))
    eft_p = jnp.pad(edge_feats.T, ((0, 2), (0, _EP - _E)))
    wp_p = jnp.pad(Wp, ((0, 2), (0, 0)))
    w1t_p = jnp.pad(We1.T, ((0, 0), (0, 2)))
    wpe_p = jnp.pad(Wpe, ((0, 2), (0, 0)))
    w2t = We2.reshape(64, 32, 32).reshape(64 * 32, 32).T
    be2t = be2.reshape(32, 32).T
    wiht = Wih.T
    whht = Whh.T
    z32 = jnp.zeros((_NP, 32), _F32)
    z64 = jnp.zeros((_NP, 64), _F32)
    zg32 = jnp.zeros((_GP, 32), _F32)
    zg64 = jnp.zeros((_GP, 64), _F32)

    h = _proj_h0(nf_p, wp_p, bp.reshape(1, 32))
    eht, eemb = _edge_pre(eft_p, ef_p, w1t_p, be1.reshape(64, 1),
                          wpe_p, bpe.reshape(1, 64))
    hidden = h
    for _ in range(3):
        hs3 = _sc_gather(h, srcs)
        msg = _msg(eht, hs3.reshape(_EP, 32), w2t, be2t)
        parts = _sc_scatter(msg.reshape(_EC, _CH, 32), dsts, z32)
        h, hidden = _gru(parts, h, hidden, wiht, whht,
                         bih.reshape(1, 96), bhh.reshape(1, 96),
                         bconv.reshape(1, 32))
    ph, pe = _sc_readout(h, eemb.reshape(_EC, _CH, 64), srcs, dsts, n2g,
                         z32, z64, zg32, zg64)
    return _head(ph, pe, W1, b1.reshape(1, 300), g1.reshape(1, 300),
                 beta1.reshape(1, 300), W2, b2.reshape(1, 256),
                 g2.reshape(1, 256), beta2.reshape(1, 256),
                 W3, b3.reshape(1, 138))


# speedup vs baseline: 1.1926x; 1.1919x over previous
import functools

import jax
import jax.numpy as jnp
from jax import lax
from jax.experimental import pallas as pl
from jax.experimental.pallas import tpu as pltpu
from jax.experimental.pallas import tpu_sc as plsc

_N, _E, _G = 10000, 160000, 512
_NP, _EP, _GP = 10240, 163840, 520
_NC, _NS = 2, 16
_NW = _NC * _NS
_CH = 128
_EC = _EP // _CH
_ECW = _EC // _NW
_KG = 20

_F32 = jnp.float32


def _sc_mesh():
    return plsc.VectorSubcoreMesh(
        core_axis_name="c", subcore_axis_name="s",
        num_cores=_NC, num_subcores=_NS)


_SC_PARAMS = pltpu.CompilerParams(use_tc_tiling_on_sc=False)


def _proj_h0_body(nf_ref, wp_ref, bp_ref, o_ref):
    o_ref[...] = jax.nn.relu(
        jnp.dot(nf_ref[...], wp_ref[...], preferred_element_type=_F32)
        + bp_ref[...])


def _proj_h0(nf_p, wp_p, bp_r):
    return pl.pallas_call(
        _proj_h0_body,
        grid=(_NP // 2048,),
        in_specs=[
            pl.BlockSpec((2048, 136), lambda i: (i, 0)),
            pl.BlockSpec((136, 32), lambda i: (0, 0)),
            pl.BlockSpec((1, 32), lambda i: (0, 0)),
        ],
        out_specs=pl.BlockSpec((2048, 32), lambda i: (i, 0)),
        out_shape=jax.ShapeDtypeStruct((_NP, 32), _F32),
    )(nf_p, wp_p, bp_r)


def _edge_pre_body(eft_ref, ef_ref, w1t_ref, b1c_ref, wpe_ref, bpe_ref,
                   eht_ref, eemb_ref):
    eht_ref[...] = jax.nn.relu(
        jnp.dot(w1t_ref[...], eft_ref[...], preferred_element_type=_F32)
        + b1c_ref[...])
    eemb_ref[...] = jax.nn.relu(
        jnp.dot(ef_ref[...], wpe_ref[...], preferred_element_type=_F32)
        + bpe_ref[...])


def _edge_pre(eft_p, ef_p, w1t_p, b1c, wpe_p, bpe_r):
    return pl.pallas_call(
        _edge_pre_body,
        grid=(_EP // 2048,),
        in_specs=[
            pl.BlockSpec((8, 2048), lambda i: (0, i)),
            pl.BlockSpec((2048, 8), lambda i: (i, 0)),
            pl.BlockSpec((64, 8), lambda i: (0, 0)),
            pl.BlockSpec((64, 1), lambda i: (0, 0)),
            pl.BlockSpec((8, 64), lambda i: (0, 0)),
            pl.BlockSpec((1, 64), lambda i: (0, 0)),
        ],
        out_specs=[
            pl.BlockSpec((64, 2048), lambda i: (0, i)),
            pl.BlockSpec((2048, 64), lambda i: (i, 0)),
        ],
        out_shape=[
            jax.ShapeDtypeStruct((64, _EP), _F32),
            jax.ShapeDtypeStruct((_EP, 64), _F32),
        ],
    )(eft_p, ef_p, w1t_p, b1c, wpe_p, bpe_r)


_BE = 1280


def _msg_body(eht_ref, hs_ref, w2t_ref, be2t_ref, msg_ref):
    ht = hs_ref[...].T
    u = (eht_ref[...][:, None, :] * ht[None, :, :]).reshape(64 * 32, _BE)
    msgt = jnp.dot(w2t_ref[...], u, preferred_element_type=_F32)
    msgt = msgt + jnp.dot(be2t_ref[...], ht, preferred_element_type=_F32)
    msg_ref[...] = msgt.T


def _msg(eht, hs, w2t, be2t):
    return pl.pallas_call(
        _msg_body,
        grid=(_EP // _BE,),
        in_specs=[
            pl.BlockSpec((64, _BE), lambda i: (0, i)),
            pl.BlockSpec((_BE, 32), lambda i: (i, 0)),
            pl.BlockSpec((32, 2048), lambda i: (0, 0)),
            pl.BlockSpec((32, 32), lambda i: (0, 0)),
        ],
        out_specs=pl.BlockSpec((_BE, 32), lambda i: (i, 0)),
        out_shape=jax.ShapeDtypeStruct((_EP, 32), _F32),
    )(eht, hs, w2t, be2t)


def _gru_body(parts_ref, h_ref, hid_ref, wih_ref, whh_ref, bih_ref, bhh_ref,
              bconv_ref, ho_ref, hido_ref):
    hid = hid_ref[...]
    m = jax.nn.relu(parts_ref[0] + parts_ref[1] + bconv_ref[...])
    gi = jnp.dot(m, wih_ref[...], preferred_element_type=_F32) + bih_ref[...]
    gh = jnp.dot(hid, whh_ref[...], preferred_element_type=_F32) + bhh_ref[...]
    r = jax.nn.sigmoid(gi[:, 0:32] + gh[:, 0:32])
    z = jax.nn.sigmoid(gi[:, 32:64] + gh[:, 32:64])
    n = jnp.tanh(gi[:, 64:96] + r * gh[:, 64:96])
    gru = (1.0 - z) * n + z * hid
    ho_ref[...] = gru + h_ref[...]
    hido_ref[...] = gru


def _gru(parts, h, hid, wiht, whht, bih_r, bhh_r, bconv_r):
    return pl.pallas_call(
        _gru_body,
        grid=(_NP // 2048,),
        in_specs=[
            pl.BlockSpec((2, 2048, 32), lambda i: (0, i, 0)),
            pl.BlockSpec((2048, 32), lambda i: (i, 0)),
            pl.BlockSpec((2048, 32), lambda i: (i, 0)),
            pl.BlockSpec((32, 96), lambda i: (0, 0)),
            pl.BlockSpec((32, 96), lambda i: (0, 0)),
            pl.BlockSpec((1, 96), lambda i: (0, 0)),
            pl.BlockSpec((1, 96), lambda i: (0, 0)),
            pl.BlockSpec((1, 32), lambda i: (0, 0)),
        ],
        out_specs=[
            pl.BlockSpec((2048, 32), lambda i: (i, 0)),
            pl.BlockSpec((2048, 32), lambda i: (i, 0)),
        ],
        out_shape=[
            jax.ShapeDtypeStruct((_NP, 32), _F32),
            jax.ShapeDtypeStruct((_NP, 32), _F32),
        ],
    )(parts, h, hid, wiht, whht, bih_r, bhh_r, bconv_r)


def _bn(x, gamma, beta):
    mu = jnp.mean(x, axis=0, keepdims=True)
    var = jnp.mean((x - mu) ** 2, axis=0, keepdims=True)
    return gamma * (x - mu) * jax.lax.rsqrt(var + 1e-5) + beta


def _head_body(ph_ref, pe_ref, w1_ref, b1_ref, g1_ref, bt1_ref,
               w2_ref, b2_ref, g2_ref, bt2_ref, w3_ref, b3_ref, o_ref):
    mol = jnp.concatenate(
        [ph_ref[0] + ph_ref[1], pe_ref[0] + pe_ref[1]], axis=1)
    mol = mol - jnp.max(mol, axis=1, keepdims=True)
    e = jnp.exp(mol)
    p = e / jnp.sum(e, axis=1, keepdims=True)
    x = jnp.dot(p, w1_ref[...], preferred_element_type=_F32) + b1_ref[...]
    x = jax.nn.relu(_bn(x, g1_ref[...], bt1_ref[...]))
    x = jnp.dot(x, w2_ref[...], preferred_element_type=_F32) + b2_ref[...]
    x = jax.nn.relu(_bn(x, g2_ref[...], bt2_ref[...]))
    o_ref[...] = jnp.dot(x, w3_ref[...], preferred_element_type=_F32) + b3_ref[...]


def _head(ph, pe, w1, b1, g1, bt1, w2, b2, g2, bt2, w3, b3):
    full = lambda s: pl.BlockSpec(s, lambda i, _s=s: tuple(0 for _ in _s))
    return pl.pallas_call(
        _head_body,
        grid=(1,),
        in_specs=[
            full((2, _G, 32)), full((2, _G, 64)),
            full((96, 300)), full((1, 300)), full((1, 300)), full((1, 300)),
            full((300, 256)), full((1, 256)), full((1, 256)), full((1, 256)),
            full((256, 138)), full((1, 138)),
        ],
        out_specs=full((_G, 138)),
        out_shape=jax.ShapeDtypeStruct((_G, 138), _F32),
    )(ph, pe, w1, b1, g1, bt1, w2, b2, g2, bt2, w3, b3)


_KB = 10


def _gather_body(h_hbm, srcs_hbm, out_hbm, idx_v, rows_v, sem, wsem, htab):
    c = lax.axis_index("c")
    s = lax.axis_index("s")
    w = s * _NC + c
    rpt = _NP // _NS
    pltpu.sync_copy(h_hbm.at[pl.ds(s * rpt, rpt)], htab.at[pl.ds(s * rpt, rpt)])
    pltpu.sync_copy(srcs_hbm.at[pl.ds(w * _ECW, _ECW)], idx_v)
    plsc.subcore_barrier()
    wr = [None, None]
    for r in range(_ECW // _KB):
        b = r % 2
        if wr[b] is not None:
            wr[b].wait()
        cps = [
            pltpu.async_copy(htab.at[idx_v.at[r * _KB + j]],
                             rows_v.at[b, j], sem)
            for j in range(_KB)
        ]
        for cp in cps:
            cp.wait()
        wr[b] = pltpu.async_copy(
            rows_v.at[b], out_hbm.at[pl.ds(w * _ECW + r * _KB, _KB)], wsem)
    for x in wr:
        x.wait()


def _sc_gather(h, srcs):
    k = functools.partial(
        pl.kernel,
        out_type=jax.ShapeDtypeStruct((_EC, _CH, 32), _F32),
        mesh=_sc_mesh(),
        compiler_params=_SC_PARAMS,
        scratch_types=[
            pltpu.VMEM((_ECW, _CH), jnp.int32),
            pltpu.VMEM((2, _KB, _CH, 32), _F32),
            pltpu.SemaphoreType.DMA,
            pltpu.SemaphoreType.DMA,
            pltpu.VMEM_SHARED((_NP, 32), _F32),
        ],
    )(_gather_body)
    return k(h, srcs)


def _scatter_body(msg3_hbm, dsts_hbm, z32_hbm, parts_hbm,
                  idx_v, rows_v, sem, acc_sh):
    c = lax.axis_index("c")
    s = lax.axis_index("s")
    w = s * _NC + c
    rpt = _NP // _NS
    pltpu.sync_copy(z32_hbm.at[pl.ds(s * rpt, rpt)],
                    acc_sh.at[pl.ds(s * rpt, rpt)])
    pltpu.sync_copy(dsts_hbm.at[pl.ds(w * _ECW, _ECW)], idx_v)
    plsc.subcore_barrier()
    for half in range(_ECW // _KG):
        pltpu.sync_copy(msg3_hbm.at[pl.ds(w * _ECW + half * _KG, _KG)],
                        rows_v)
        cps = [
            pltpu.async_copy(rows_v.at[j],
                             acc_sh.at[idx_v.at[half * _KG + j]],
                             sem, add=True)
            for j in range(_KG)
        ]
        for cp in cps:
            cp.wait()
    plsc.subcore_barrier()
    pltpu.sync_copy(acc_sh.at[pl.ds(s * rpt, rpt)],
                    parts_hbm.at[c, pl.ds(s * rpt, rpt)])


def _sc_scatter(msg3, dsts, z32):
    k = functools.partial(
        pl.kernel,
        out_type=jax.ShapeDtypeStruct((_NC, _NP, 32), _F32),
        mesh=_sc_mesh(),
        compiler_params=_SC_PARAMS,
        scratch_types=[
            pltpu.VMEM((_ECW, _CH), jnp.int32),
            pltpu.VMEM((_KG, _CH, 32), _F32),
            pltpu.SemaphoreType.DMA,
            pltpu.VMEM_SHARED((_NP, 32), _F32),
        ],
    )(_scatter_body)
    return k(msg3, dsts, z32)


_KR = 4


def _readout_body(h_hbm, eemb3_hbm, srcs_hbm, dsts_hbm, n2g_hbm,
                  z32_hbm, z64_hbm, zg32_hbm, zg64_hbm,
                  outh_hbm, oute_hbm,
                  idxs_v, idxd_v, idxn_v, rows32_v, rows64_v, sem,
                  acc_h, acc_e, acc_gh, acc_ge):
    c = lax.axis_index("c")
    s = lax.axis_index("s")
    w = s * _NC + c
    rpt = _NP // _NS
    gpt = _G // _NS
    npt = (_NP // _CH) // _NS
    pltpu.sync_copy(z32_hbm.at[pl.ds(s * rpt, rpt)],
                    acc_h.at[pl.ds(s * rpt, rpt)])
    pltpu.sync_copy(z64_hbm.at[pl.ds(s * rpt, rpt)],
                    acc_e.at[pl.ds(s * rpt, rpt)])
    pltpu.sync_copy(zg32_hbm.at[pl.ds(s * gpt, gpt)],
                    acc_gh.at[pl.ds(s * gpt, gpt)])
    pltpu.sync_copy(zg64_hbm.at[pl.ds(s * gpt, gpt)],
                    acc_ge.at[pl.ds(s * gpt, gpt)])
    @pl.when(s == 0)
    def _():
        pltpu.sync_copy(zg32_hbm.at[pl.ds(_G, _GP - _G)],
                        acc_gh.at[pl.ds(_G, _GP - _G)])
        pltpu.sync_copy(zg64_hbm.at[pl.ds(_G, _GP - _G)],
                        acc_ge.at[pl.ds(_G, _GP - _G)])
    pltpu.sync_copy(srcs_hbm.at[pl.ds(w * _ECW, _ECW)], idxs_v)
    pltpu.sync_copy(dsts_hbm.at[pl.ds(w * _ECW, _ECW)], idxd_v)
    plsc.subcore_barrier()
    for r in range(_ECW // _KR):
        cps = [
            pltpu.async_copy(h_hbm.at[idxs_v.at[r * _KR + j]],
                             rows32_v.at[j], sem)
            for j in range(_KR)
        ]
        pltpu.sync_copy(eemb3_hbm.at[pl.ds(w * _ECW + r * _KR, _KR)],
                        rows64_v)
        for cp in cps:
            cp.wait()
        cps = [
            pltpu.async_copy(rows32_v.at[j],
                             acc_h.at[idxd_v.at[r * _KR + j]],
                             sem, add=True)
            for j in range(_KR)
        ] + [
            pltpu.async_copy(rows64_v.at[j],
                             acc_e.at[idxd_v.at[r * _KR + j]],
                             sem, add=True)
            for j in range(_KR)
        ]
        for cp in cps:
            cp.wait()
    plsc.subcore_barrier()
    pltpu.sync_copy(n2g_hbm.at[pl.ds(s * npt, npt)], idxn_v)
    for t in range(npt):
        p = s * npt + t
        pltpu.sync_copy(acc_h.at[pl.ds(p * _CH, _CH)], rows32_v.at[0])
        pltpu.sync_copy(acc_e.at[pl.ds(p * _CH, _CH)], rows64_v.at[0])
        pltpu.sync_copy(rows32_v.at[0], acc_gh.at[idxn_v.at[t]], add=True)
        pltpu.sync_copy(rows64_v.at[0], acc_ge.at[idxn_v.at[t]], add=True)
    plsc.subcore_barrier()
    pltpu.sync_copy(acc_gh.at[pl.ds(s * gpt, gpt)],
                    outh_hbm.at[c, pl.ds(s * gpt, gpt)])
    pltpu.sync_copy(acc_ge.at[pl.ds(s * gpt, gpt)],
                    oute_hbm.at[c, pl.ds(s * gpt, gpt)])


def _sc_readout(h, eemb3, srcs, dsts, n2g, z32, z64, zg32, zg64):
    k = functools.partial(
        pl.kernel,
        out_type=(
            jax.ShapeDtypeStruct((_NC, _G, 32), _F32),
            jax.ShapeDtypeStruct((_NC, _G, 64), _F32),
        ),
        mesh=_sc_mesh(),
        compiler_params=_SC_PARAMS,
        scratch_types=[
            pltpu.VMEM((_ECW, _CH), jnp.int32),
            pltpu.VMEM((_ECW, _CH), jnp.int32),
            pltpu.VMEM(((_NP // _CH) // _NS, _CH), jnp.int32),
            pltpu.VMEM((_KR, _CH, 32), _F32),
            pltpu.VMEM((_KR, _CH, 64), _F32),
            pltpu.SemaphoreType.DMA,
            pltpu.VMEM_SHARED((_NP, 32), _F32),
            pltpu.VMEM_SHARED((_NP, 64), _F32),
            pltpu.VMEM_SHARED((_GP, 32), _F32),
            pltpu.VMEM_SHARED((_GP, 64), _F32),
        ],
    )(_readout_body)
    return k(h, eemb3, srcs, dsts, n2g, z32, z64, zg32, zg64)


def kernel(node_feats, edge_feats, edge_index, node2graph, Wp, bp, We1, be1,
           We2, be2, bconv, Wih, Whh, bih, bhh, Wpe, bpe, W1, b1, g1, beta1,
           W2, b2, g2, beta2, W3, b3):
    i32 = jnp.int32
    src = edge_index[0]
    dst = edge_index[1]
    srcs = jnp.concatenate(
        [src, jnp.zeros((_EP - _E,), i32)]).reshape(_EC, _CH)
    dsts = jnp.concatenate(
        [dst, jnp.full((_EP - _E,), _N, i32)]).reshape(_EC, _CH)
    n2g = jnp.concatenate(
        [node2graph, jnp.full((_NP - _N,), _G, i32)]).reshape(_NP // _CH, _CH)
    nf_p = jnp.pad(node_feats, ((0, _NP - _N), (0, 2)))
    ef_p = jnp.pad(edge_feats, ((0, _EP - _E), (0, 2)))
    eft_p = jnp.pad(edge_feats.T, ((0, 2), (0, _EP - _E)))
    wp_p = jnp.pad(Wp, ((0, 2), (0, 0)))
    w1t_p = jnp.pad(We1.T, ((0, 0), (0, 2)))
    wpe_p = jnp.pad(Wpe, ((0, 2), (0, 0)))
    w2t = We2.reshape(64, 32, 32).reshape(64 * 32, 32).T
    be2t = be2.reshape(32, 32).T
    wiht = Wih.T
    whht = Whh.T
    z32 = jnp.zeros((_NP, 32), _F32)
    z64 = jnp.zeros((_NP, 64), _F32)
    zg32 = jnp.zeros((_GP, 32), _F32)
    zg64 = jnp.zeros((_GP, 64), _F32)

    h = _proj_h0(nf_p, wp_p, bp.reshape(1, 32))
    eht, eemb = _edge_pre(eft_p, ef_p, w1t_p, be1.reshape(64, 1),
                          wpe_p, bpe.reshape(1, 64))
    hidden = h
    for _ in range(3):
        hs3 = _sc_gather(h, srcs)
        msg = _msg(eht, hs3.reshape(_EP, 32), w2t, be2t)
        parts = _sc_scatter(msg.reshape(_EC, _CH, 32), dsts, z32)
        h, hidden = _gru(parts, h, hidden, wiht, whht,
                         bih.reshape(1, 96), bhh.reshape(1, 96),
                         bconv.reshape(1, 32))
    ph, pe = _sc_readout(h, eemb.reshape(_EC, _CH, 64), srcs, dsts, n2g,
                         z32, z64, zg32, zg64)
    return _head(ph, pe, W1, b1.reshape(1, 300), g1.reshape(1, 300),
                 beta1.reshape(1, 300), W2, b2.reshape(1, 256),
                 g2.reshape(1, 256), beta2.reshape(1, 256),
                 W3, b3.reshape(1, 138))


# speedup vs baseline: 1.2541x; 1.0516x over previous
import functools

import jax
import jax.numpy as jnp
from jax import lax
from jax.experimental import pallas as pl
from jax.experimental.pallas import tpu as pltpu
from jax.experimental.pallas import tpu_sc as plsc

_N, _E, _G = 10000, 160000, 512
_NP, _EP, _GP = 10240, 163840, 520
_NC, _NS = 2, 16
_NW = _NC * _NS
_CH = 128
_EC = _EP // _CH
_ECW = _EC // _NW
_KG = 20

_F32 = jnp.float32


def _sc_mesh():
    return plsc.VectorSubcoreMesh(
        core_axis_name="c", subcore_axis_name="s",
        num_cores=_NC, num_subcores=_NS)


_SC_PARAMS = pltpu.CompilerParams(use_tc_tiling_on_sc=False)


def _proj_h0_body(nf_ref, wp_ref, bp_ref, o_ref):
    o_ref[...] = jax.nn.relu(
        jnp.dot(nf_ref[...], wp_ref[...], preferred_element_type=_F32)
        + bp_ref[...])


def _proj_h0(nf_p, wp_p, bp_r):
    return pl.pallas_call(
        _proj_h0_body,
        grid=(_NP // 2048,),
        in_specs=[
            pl.BlockSpec((2048, 136), lambda i: (i, 0)),
            pl.BlockSpec((136, 32), lambda i: (0, 0)),
            pl.BlockSpec((1, 32), lambda i: (0, 0)),
        ],
        out_specs=pl.BlockSpec((2048, 32), lambda i: (i, 0)),
        out_shape=jax.ShapeDtypeStruct((_NP, 32), _F32),
    )(nf_p, wp_p, bp_r)


def _edge_pre_body(eft_ref, ef_ref, w1t_ref, b1c_ref, wpe_ref, bpe_ref,
                   eht_ref, eemb_ref):
    eht_ref[...] = jax.nn.relu(
        jnp.dot(w1t_ref[...], eft_ref[...], preferred_element_type=_F32)
        + b1c_ref[...])
    eemb_ref[...] = jax.nn.relu(
        jnp.dot(ef_ref[...], wpe_ref[...], preferred_element_type=_F32)
        + bpe_ref[...])


def _edge_pre(eft_p, ef_p, w1t_p, b1c, wpe_p, bpe_r):
    return pl.pallas_call(
        _edge_pre_body,
        grid=(_EP // 2048,),
        in_specs=[
            pl.BlockSpec((8, 2048), lambda i: (0, i)),
            pl.BlockSpec((2048, 8), lambda i: (i, 0)),
            pl.BlockSpec((64, 8), lambda i: (0, 0)),
            pl.BlockSpec((64, 1), lambda i: (0, 0)),
            pl.BlockSpec((8, 64), lambda i: (0, 0)),
            pl.BlockSpec((1, 64), lambda i: (0, 0)),
        ],
        out_specs=[
            pl.BlockSpec((64, 2048), lambda i: (0, i)),
            pl.BlockSpec((2048, 64), lambda i: (i, 0)),
        ],
        out_shape=[
            jax.ShapeDtypeStruct((64, _EP), _F32),
            jax.ShapeDtypeStruct((_EP, 64), _F32),
        ],
    )(eft_p, ef_p, w1t_p, b1c, wpe_p, bpe_r)


_BE = 2560


def _msg_body(eht_ref, hs_ref, w2t_ref, be2t_ref, msg_ref):
    ht = hs_ref[...].T
    u = (eht_ref[...][:, None, :] * ht[None, :, :]).reshape(64 * 32, _BE)
    msgt = jnp.dot(w2t_ref[...], u, preferred_element_type=_F32)
    msgt = msgt + jnp.dot(be2t_ref[...], ht, preferred_element_type=_F32)
    msg_ref[...] = msgt.T


def _msg(eht, hs, w2t, be2t):
    return pl.pallas_call(
        _msg_body,
        grid=(_EP // _BE,),
        in_specs=[
            pl.BlockSpec((64, _BE), lambda i: (0, i)),
            pl.BlockSpec((_BE, 32), lambda i: (i, 0)),
            pl.BlockSpec((32, 2048), lambda i: (0, 0)),
            pl.BlockSpec((32, 32), lambda i: (0, 0)),
        ],
        out_specs=pl.BlockSpec((_BE, 32), lambda i: (i, 0)),
        out_shape=jax.ShapeDtypeStruct((_EP, 32), _F32),
    )(eht, hs, w2t, be2t)


def _gru_body(parts_ref, h_ref, hid_ref, wih_ref, whh_ref, bih_ref, bhh_ref,
              bconv_ref, ho_ref, hido_ref):
    hid = hid_ref[...]
    m = jax.nn.relu(parts_ref[0] + parts_ref[1] + bconv_ref[...])
    gi = jnp.dot(m, wih_ref[...], preferred_element_type=_F32) + bih_ref[...]
    gh = jnp.dot(hid, whh_ref[...], preferred_element_type=_F32) + bhh_ref[...]
    r = jax.nn.sigmoid(gi[:, 0:32] + gh[:, 0:32])
    z = jax.nn.sigmoid(gi[:, 32:64] + gh[:, 32:64])
    n = jnp.tanh(gi[:, 64:96] + r * gh[:, 64:96])
    gru = (1.0 - z) * n + z * hid
    ho_ref[...] = gru + h_ref[...]
    hido_ref[...] = gru


def _gru(parts, h, hid, wiht, whht, bih_r, bhh_r, bconv_r):
    return pl.pallas_call(
        _gru_body,
        grid=(_NP // 2048,),
        in_specs=[
            pl.BlockSpec((2, 2048, 32), lambda i: (0, i, 0)),
            pl.BlockSpec((2048, 32), lambda i: (i, 0)),
            pl.BlockSpec((2048, 32), lambda i: (i, 0)),
            pl.BlockSpec((32, 96), lambda i: (0, 0)),
            pl.BlockSpec((32, 96), lambda i: (0, 0)),
            pl.BlockSpec((1, 96), lambda i: (0, 0)),
            pl.BlockSpec((1, 96), lambda i: (0, 0)),
            pl.BlockSpec((1, 32), lambda i: (0, 0)),
        ],
        out_specs=[
            pl.BlockSpec((2048, 32), lambda i: (i, 0)),
            pl.BlockSpec((2048, 32), lambda i: (i, 0)),
        ],
        out_shape=[
            jax.ShapeDtypeStruct((_NP, 32), _F32),
            jax.ShapeDtypeStruct((_NP, 32), _F32),
        ],
    )(parts, h, hid, wiht, whht, bih_r, bhh_r, bconv_r)


def _bn(x, gamma, beta):
    mu = jnp.mean(x, axis=0, keepdims=True)
    var = jnp.mean((x - mu) ** 2, axis=0, keepdims=True)
    return gamma * (x - mu) * jax.lax.rsqrt(var + 1e-5) + beta


def _head_body(ph_ref, pe_ref, w1_ref, b1_ref, g1_ref, bt1_ref,
               w2_ref, b2_ref, g2_ref, bt2_ref, w3_ref, b3_ref, o_ref):
    mol = jnp.concatenate(
        [ph_ref[0] + ph_ref[1], pe_ref[0] + pe_ref[1]], axis=1)
    mol = mol - jnp.max(mol, axis=1, keepdims=True)
    e = jnp.exp(mol)
    p = e / jnp.sum(e, axis=1, keepdims=True)
    x = jnp.dot(p, w1_ref[...], preferred_element_type=_F32) + b1_ref[...]
    x = jax.nn.relu(_bn(x, g1_ref[...], bt1_ref[...]))
    x = jnp.dot(x, w2_ref[...], preferred_element_type=_F32) + b2_ref[...]
    x = jax.nn.relu(_bn(x, g2_ref[...], bt2_ref[...]))
    o_ref[...] = jnp.dot(x, w3_ref[...], preferred_element_type=_F32) + b3_ref[...]


def _head(ph, pe, w1, b1, g1, bt1, w2, b2, g2, bt2, w3, b3):
    full = lambda s: pl.BlockSpec(s, lambda i, _s=s: tuple(0 for _ in _s))
    return pl.pallas_call(
        _head_body,
        grid=(1,),
        in_specs=[
            full((2, _G, 32)), full((2, _G, 64)),
            full((96, 300)), full((1, 300)), full((1, 300)), full((1, 300)),
            full((300, 256)), full((1, 256)), full((1, 256)), full((1, 256)),
            full((256, 138)), full((1, 138)),
        ],
        out_specs=full((_G, 138)),
        out_shape=jax.ShapeDtypeStruct((_G, 138), _F32),
    )(ph, pe, w1, b1, g1, bt1, w2, b2, g2, bt2, w3, b3)


_KB = 10


def _gather_body(h_hbm, srcs_hbm, out_hbm, idx_v, rows_v, sem, wsem, htab):
    c = lax.axis_index("c")
    s = lax.axis_index("s")
    w = s * _NC + c
    rpt = _NP // _NS
    pltpu.sync_copy(h_hbm.at[pl.ds(s * rpt, rpt)], htab.at[pl.ds(s * rpt, rpt)])
    pltpu.sync_copy(srcs_hbm.at[pl.ds(w * _ECW, _ECW)], idx_v)
    plsc.subcore_barrier()
    wr = [None, None]
    for r in range(_ECW // _KB):
        b = r % 2
        if wr[b] is not None:
            wr[b].wait()
        cps = [
            pltpu.async_copy(htab.at[idx_v.at[r * _KB + j]],
                             rows_v.at[b, j], sem)
            for j in range(_KB)
        ]
        for cp in cps:
            cp.wait()
        wr[b] = pltpu.async_copy(
            rows_v.at[b], out_hbm.at[pl.ds(w * _ECW + r * _KB, _KB)], wsem)
    for x in wr:
        x.wait()


def _sc_gather(h, srcs):
    k = functools.partial(
        pl.kernel,
        out_type=jax.ShapeDtypeStruct((_EC, _CH, 32), _F32),
        mesh=_sc_mesh(),
        compiler_params=_SC_PARAMS,
        scratch_types=[
            pltpu.VMEM((_ECW, _CH), jnp.int32),
            pltpu.VMEM((2, _KB, _CH, 32), _F32),
            pltpu.SemaphoreType.DMA,
            pltpu.SemaphoreType.DMA,
            pltpu.VMEM_SHARED((_NP, 32), _F32),
        ],
    )(_gather_body)
    return k(h, srcs)


def _scatter_body(msg3_hbm, dsts_hbm, z32_hbm, parts_hbm,
                  idx_v, rows_v, sem, acc_sh):
    c = lax.axis_index("c")
    s = lax.axis_index("s")
    w = s * _NC + c
    rpt = _NP // _NS
    pltpu.sync_copy(z32_hbm.at[pl.ds(s * rpt, rpt)],
                    acc_sh.at[pl.ds(s * rpt, rpt)])
    pltpu.sync_copy(dsts_hbm.at[pl.ds(w * _ECW, _ECW)], idx_v)
    plsc.subcore_barrier()
    for half in range(_ECW // _KG):
        pltpu.sync_copy(msg3_hbm.at[pl.ds(w * _ECW + half * _KG, _KG)],
                        rows_v)
        cps = [
            pltpu.async_copy(rows_v.at[j],
                             acc_sh.at[idx_v.at[half * _KG + j]],
                             sem, add=True)
            for j in range(_KG)
        ]
        for cp in cps:
            cp.wait()
    plsc.subcore_barrier()
    pltpu.sync_copy(acc_sh.at[pl.ds(s * rpt, rpt)],
                    parts_hbm.at[c, pl.ds(s * rpt, rpt)])


def _sc_scatter(msg3, dsts, z32):
    k = functools.partial(
        pl.kernel,
        out_type=jax.ShapeDtypeStruct((_NC, _NP, 32), _F32),
        mesh=_sc_mesh(),
        compiler_params=_SC_PARAMS,
        scratch_types=[
            pltpu.VMEM((_ECW, _CH), jnp.int32),
            pltpu.VMEM((_KG, _CH, 32), _F32),
            pltpu.SemaphoreType.DMA,
            pltpu.VMEM_SHARED((_NP, 32), _F32),
        ],
    )(_scatter_body)
    return k(msg3, dsts, z32)


_KR = 4


def _readout_body(h_hbm, eemb3_hbm, srcs_hbm, dsts_hbm, n2g_hbm,
                  z32_hbm, z64_hbm, zg32_hbm, zg64_hbm,
                  outh_hbm, oute_hbm,
                  idxs_v, idxd_v, idxn_v, rows32_v, rows64_v, sem,
                  acc_h, acc_e, acc_gh, acc_ge):
    c = lax.axis_index("c")
    s = lax.axis_index("s")
    w = s * _NC + c
    rpt = _NP // _NS
    gpt = _G // _NS
    npt = (_NP // _CH) // _NS
    pltpu.sync_copy(z32_hbm.at[pl.ds(s * rpt, rpt)],
                    acc_h.at[pl.ds(s * rpt, rpt)])
    pltpu.sync_copy(z64_hbm.at[pl.ds(s * rpt, rpt)],
                    acc_e.at[pl.ds(s * rpt, rpt)])
    pltpu.sync_copy(zg32_hbm.at[pl.ds(s * gpt, gpt)],
                    acc_gh.at[pl.ds(s * gpt, gpt)])
    pltpu.sync_copy(zg64_hbm.at[pl.ds(s * gpt, gpt)],
                    acc_ge.at[pl.ds(s * gpt, gpt)])
    @pl.when(s == 0)
    def _():
        pltpu.sync_copy(zg32_hbm.at[pl.ds(_G, _GP - _G)],
                        acc_gh.at[pl.ds(_G, _GP - _G)])
        pltpu.sync_copy(zg64_hbm.at[pl.ds(_G, _GP - _G)],
                        acc_ge.at[pl.ds(_G, _GP - _G)])
    pltpu.sync_copy(srcs_hbm.at[pl.ds(w * _ECW, _ECW)], idxs_v)
    pltpu.sync_copy(dsts_hbm.at[pl.ds(w * _ECW, _ECW)], idxd_v)
    plsc.subcore_barrier()
    for r in range(_ECW // _KR):
        cps = [
            pltpu.async_copy(h_hbm.at[idxs_v.at[r * _KR + j]],
                             rows32_v.at[j], sem)
            for j in range(_KR)
        ]
        pltpu.sync_copy(eemb3_hbm.at[pl.ds(w * _ECW + r * _KR, _KR)],
                        rows64_v)
        for cp in cps:
            cp.wait()
        cps = [
            pltpu.async_copy(rows32_v.at[j],
                             acc_h.at[idxd_v.at[r * _KR + j]],
                             sem, add=True)
            for j in range(_KR)
        ] + [
            pltpu.async_copy(rows64_v.at[j],
                             acc_e.at[idxd_v.at[r * _KR + j]],
                             sem, add=True)
            for j in range(_KR)
        ]
        for cp in cps:
            cp.wait()
    plsc.subcore_barrier()
    pltpu.sync_copy(n2g_hbm.at[pl.ds(s * npt, npt)], idxn_v)
    for t in range(npt):
        p = s * npt + t
        pltpu.sync_copy(acc_h.at[pl.ds(p * _CH, _CH)], rows32_v.at[0])
        pltpu.sync_copy(acc_e.at[pl.ds(p * _CH, _CH)], rows64_v.at[0])
        pltpu.sync_copy(rows32_v.at[0], acc_gh.at[idxn_v.at[t]], add=True)
        pltpu.sync_copy(rows64_v.at[0], acc_ge.at[idxn_v.at[t]], add=True)
    plsc.subcore_barrier()
    pltpu.sync_copy(acc_gh.at[pl.ds(s * gpt, gpt)],
                    outh_hbm.at[c, pl.ds(s * gpt, gpt)])
    pltpu.sync_copy(acc_ge.at[pl.ds(s * gpt, gpt)],
                    oute_hbm.at[c, pl.ds(s * gpt, gpt)])


def _sc_readout(h, eemb3, srcs, dsts, n2g, z32, z64, zg32, zg64):
    k = functools.partial(
        pl.kernel,
        out_type=(
            jax.ShapeDtypeStruct((_NC, _G, 32), _F32),
            jax.ShapeDtypeStruct((_NC, _G, 64), _F32),
        ),
        mesh=_sc_mesh(),
        compiler_params=_SC_PARAMS,
        scratch_types=[
            pltpu.VMEM((_ECW, _CH), jnp.int32),
            pltpu.VMEM((_ECW, _CH), jnp.int32),
            pltpu.VMEM(((_NP // _CH) // _NS, _CH), jnp.int32),
            pltpu.VMEM((_KR, _CH, 32), _F32),
            pltpu.VMEM((_KR, _CH, 64), _F32),
            pltpu.SemaphoreType.DMA,
            pltpu.VMEM_SHARED((_NP, 32), _F32),
            pltpu.VMEM_SHARED((_NP, 64), _F32),
            pltpu.VMEM_SHARED((_GP, 32), _F32),
            pltpu.VMEM_SHARED((_GP, 64), _F32),
        ],
    )(_readout_body)
    return k(h, eemb3, srcs, dsts, n2g, z32, z64, zg32, zg64)


def kernel(node_feats, edge_feats, edge_index, node2graph, Wp, bp, We1, be1,
           We2, be2, bconv, Wih, Whh, bih, bhh, Wpe, bpe, W1, b1, g1, beta1,
           W2, b2, g2, beta2, W3, b3):
    i32 = jnp.int32
    src = edge_index[0]
    dst = edge_index[1]
    srcs = jnp.concatenate(
        [src, jnp.zeros((_EP - _E,), i32)]).reshape(_EC, _CH)
    dsts = jnp.concatenate(
        [dst, jnp.full((_EP - _E,), _N, i32)]).reshape(_EC, _CH)
    n2g = jnp.concatenate(
        [node2graph, jnp.full((_NP - _N,), _G, i32)]).reshape(_NP // _CH, _CH)
    nf_p = jnp.pad(node_feats, ((0, _NP - _N), (0, 2)))
    ef_p = jnp.pad(edge_feats, ((0, _EP - _E), (0, 2)))
    eft_p = jnp.pad(edge_feats.T, ((0, 2), (0, _EP - _E)))
    wp_p = jnp.pad(Wp, ((0, 2), (0, 0)))
    w1t_p = jnp.pad(We1.T, ((0, 0), (0, 2)))
    wpe_p = jnp.pad(Wpe, ((0, 2), (0, 0)))
    w2t = We2.reshape(64, 32, 32).reshape(64 * 32, 32).T
    be2t = be2.reshape(32, 32).T
    wiht = Wih.T
    whht = Whh.T
    z32 = jnp.zeros((_NP, 32), _F32)
    z64 = jnp.zeros((_NP, 64), _F32)
    zg32 = jnp.zeros((_GP, 32), _F32)
    zg64 = jnp.zeros((_GP, 64), _F32)

    h = _proj_h0(nf_p, wp_p, bp.reshape(1, 32))
    eht, eemb = _edge_pre(eft_p, ef_p, w1t_p, be1.reshape(64, 1),
                          wpe_p, bpe.reshape(1, 64))
    hidden = h
    for _ in range(3):
        hs3 = _sc_gather(h, srcs)
        msg = _msg(eht, hs3.reshape(_EP, 32), w2t, be2t)
        parts = _sc_scatter(msg.reshape(_EC, _CH, 32), dsts, z32)
        h, hidden = _gru(parts, h, hidden, wiht, whht,
                         bih.reshape(1, 96), bhh.reshape(1, 96),
                         bconv.reshape(1, 32))
    ph, pe = _sc_readout(h, eemb.reshape(_EC, _CH, 64), srcs, dsts, n2g,
                         z32, z64, zg32, zg64)
    return _head(ph, pe, W1, b1.reshape(1, 300), g1.reshape(1, 300),
                 beta1.reshape(1, 300), W2, b2.reshape(1, 256),
                 g2.reshape(1, 256), beta2.reshape(1, 256),
                 W3, b3.reshape(1, 138))


# speedup vs baseline: 1.2764x; 1.0177x over previous
import functools

import jax
import jax.numpy as jnp
from jax import lax
from jax.experimental import pallas as pl
from jax.experimental.pallas import tpu as pltpu
from jax.experimental.pallas import tpu_sc as plsc

_N, _E, _G = 10000, 160000, 512
_NP, _EP, _GP = 10240, 163840, 520
_NC, _NS = 2, 16
_NW = _NC * _NS
_CH = 128
_EC = _EP // _CH
_ECW = _EC // _NW
_KG = 20

_F32 = jnp.float32


def _sc_mesh():
    return plsc.VectorSubcoreMesh(
        core_axis_name="c", subcore_axis_name="s",
        num_cores=_NC, num_subcores=_NS)


_SC_PARAMS = pltpu.CompilerParams(use_tc_tiling_on_sc=False)


def _proj_h0_body(nf_ref, wp_ref, bp_ref, o_ref):
    o_ref[...] = jax.nn.relu(
        jnp.dot(nf_ref[...], wp_ref[...], preferred_element_type=_F32)
        + bp_ref[...])


def _proj_h0(nf_p, wp_p, bp_r):
    return pl.pallas_call(
        _proj_h0_body,
        grid=(_NP // 2048,),
        in_specs=[
            pl.BlockSpec((2048, 136), lambda i: (i, 0)),
            pl.BlockSpec((136, 32), lambda i: (0, 0)),
            pl.BlockSpec((1, 32), lambda i: (0, 0)),
        ],
        out_specs=pl.BlockSpec((2048, 32), lambda i: (i, 0)),
        out_shape=jax.ShapeDtypeStruct((_NP, 32), _F32),
    )(nf_p, wp_p, bp_r)


def _edge_pre_body(eft_ref, ef_ref, w1t_ref, b1c_ref, wpe_ref, bpe_ref,
                   eht_ref, eemb_ref):
    eht_ref[...] = jax.nn.relu(
        jnp.dot(w1t_ref[...], eft_ref[...], preferred_element_type=_F32)
        + b1c_ref[...])
    eemb_ref[...] = jax.nn.relu(
        jnp.dot(ef_ref[...], wpe_ref[...], preferred_element_type=_F32)
        + bpe_ref[...])


def _edge_pre(eft_p, ef_p, w1t_p, b1c, wpe_p, bpe_r):
    return pl.pallas_call(
        _edge_pre_body,
        grid=(_EP // 2048,),
        in_specs=[
            pl.BlockSpec((8, 2048), lambda i: (0, i)),
            pl.BlockSpec((2048, 8), lambda i: (i, 0)),
            pl.BlockSpec((64, 8), lambda i: (0, 0)),
            pl.BlockSpec((64, 1), lambda i: (0, 0)),
            pl.BlockSpec((8, 64), lambda i: (0, 0)),
            pl.BlockSpec((1, 64), lambda i: (0, 0)),
        ],
        out_specs=[
            pl.BlockSpec((64, 2048), lambda i: (0, i)),
            pl.BlockSpec((2048, 64), lambda i: (i, 0)),
        ],
        out_shape=[
            jax.ShapeDtypeStruct((64, _EP), _F32),
            jax.ShapeDtypeStruct((_EP, 64), _F32),
        ],
    )(eft_p, ef_p, w1t_p, b1c, wpe_p, bpe_r)


_BE = 5120


def _msg_body(eht_ref, hs_ref, w2t_ref, be2t_ref, msg_ref):
    ht = hs_ref[...].T
    u = (eht_ref[...][:, None, :] * ht[None, :, :]).reshape(64 * 32, _BE)
    msgt = jnp.dot(w2t_ref[...], u, preferred_element_type=_F32)
    msgt = msgt + jnp.dot(be2t_ref[...], ht, preferred_element_type=_F32)
    msg_ref[...] = msgt.T


def _msg(eht, hs, w2t, be2t):
    return pl.pallas_call(
        _msg_body,
        grid=(_EP // _BE,),
        in_specs=[
            pl.BlockSpec((64, _BE), lambda i: (0, i)),
            pl.BlockSpec((_BE, 32), lambda i: (i, 0)),
            pl.BlockSpec((32, 2048), lambda i: (0, 0)),
            pl.BlockSpec((32, 32), lambda i: (0, 0)),
        ],
        out_specs=pl.BlockSpec((_BE, 32), lambda i: (i, 0)),
        out_shape=jax.ShapeDtypeStruct((_EP, 32), _F32),
    )(eht, hs, w2t, be2t)


def _gru_body(parts_ref, h_ref, hid_ref, wih_ref, whh_ref, bih_ref, bhh_ref,
              bconv_ref, ho_ref, hido_ref):
    hid = hid_ref[...]
    m = jax.nn.relu(parts_ref[0] + parts_ref[1] + bconv_ref[...])
    gi = jnp.dot(m, wih_ref[...], preferred_element_type=_F32) + bih_ref[...]
    gh = jnp.dot(hid, whh_ref[...], preferred_element_type=_F32) + bhh_ref[...]
    r = jax.nn.sigmoid(gi[:, 0:32] + gh[:, 0:32])
    z = jax.nn.sigmoid(gi[:, 32:64] + gh[:, 32:64])
    n = jnp.tanh(gi[:, 64:96] + r * gh[:, 64:96])
    gru = (1.0 - z) * n + z * hid
    ho_ref[...] = gru + h_ref[...]
    hido_ref[...] = gru


def _gru(parts, h, hid, wiht, whht, bih_r, bhh_r, bconv_r):
    return pl.pallas_call(
        _gru_body,
        grid=(_NP // 2048,),
        in_specs=[
            pl.BlockSpec((2, 2048, 32), lambda i: (0, i, 0)),
            pl.BlockSpec((2048, 32), lambda i: (i, 0)),
            pl.BlockSpec((2048, 32), lambda i: (i, 0)),
            pl.BlockSpec((32, 96), lambda i: (0, 0)),
            pl.BlockSpec((32, 96), lambda i: (0, 0)),
            pl.BlockSpec((1, 96), lambda i: (0, 0)),
            pl.BlockSpec((1, 96), lambda i: (0, 0)),
            pl.BlockSpec((1, 32), lambda i: (0, 0)),
        ],
        out_specs=[
            pl.BlockSpec((2048, 32), lambda i: (i, 0)),
            pl.BlockSpec((2048, 32), lambda i: (i, 0)),
        ],
        out_shape=[
            jax.ShapeDtypeStruct((_NP, 32), _F32),
            jax.ShapeDtypeStruct((_NP, 32), _F32),
        ],
    )(parts, h, hid, wiht, whht, bih_r, bhh_r, bconv_r)


def _bn(x, gamma, beta):
    mu = jnp.mean(x, axis=0, keepdims=True)
    var = jnp.mean((x - mu) ** 2, axis=0, keepdims=True)
    return gamma * (x - mu) * jax.lax.rsqrt(var + 1e-5) + beta


def _head_body(ph_ref, pe_ref, w1_ref, b1_ref, g1_ref, bt1_ref,
               w2_ref, b2_ref, g2_ref, bt2_ref, w3_ref, b3_ref, o_ref):
    mol = jnp.concatenate(
        [ph_ref[0] + ph_ref[1], pe_ref[0] + pe_ref[1]], axis=1)
    mol = mol - jnp.max(mol, axis=1, keepdims=True)
    e = jnp.exp(mol)
    p = e / jnp.sum(e, axis=1, keepdims=True)
    x = jnp.dot(p, w1_ref[...], preferred_element_type=_F32) + b1_ref[...]
    x = jax.nn.relu(_bn(x, g1_ref[...], bt1_ref[...]))
    x = jnp.dot(x, w2_ref[...], preferred_element_type=_F32) + b2_ref[...]
    x = jax.nn.relu(_bn(x, g2_ref[...], bt2_ref[...]))
    o_ref[...] = jnp.dot(x, w3_ref[...], preferred_element_type=_F32) + b3_ref[...]


def _head(ph, pe, w1, b1, g1, bt1, w2, b2, g2, bt2, w3, b3):
    full = lambda s: pl.BlockSpec(s, lambda i, _s=s: tuple(0 for _ in _s))
    return pl.pallas_call(
        _head_body,
        grid=(1,),
        in_specs=[
            full((2, _G, 32)), full((2, _G, 64)),
            full((96, 300)), full((1, 300)), full((1, 300)), full((1, 300)),
            full((300, 256)), full((1, 256)), full((1, 256)), full((1, 256)),
            full((256, 138)), full((1, 138)),
        ],
        out_specs=full((_G, 138)),
        out_shape=jax.ShapeDtypeStruct((_G, 138), _F32),
    )(ph, pe, w1, b1, g1, bt1, w2, b2, g2, bt2, w3, b3)


_KB = 10


def _gather_body(h_hbm, srcs_hbm, out_hbm, idx_v, rows_v, sem, wsem, htab):
    c = lax.axis_index("c")
    s = lax.axis_index("s")
    w = s * _NC + c
    rpt = _NP // _NS
    pltpu.sync_copy(h_hbm.at[pl.ds(s * rpt, rpt)], htab.at[pl.ds(s * rpt, rpt)])
    pltpu.sync_copy(srcs_hbm.at[pl.ds(w * _ECW, _ECW)], idx_v)
    plsc.subcore_barrier()
    wr = [None, None]
    for r in range(_ECW // _KB):
        b = r % 2
        if wr[b] is not None:
            wr[b].wait()
        cps = [
            pltpu.async_copy(htab.at[idx_v.at[r * _KB + j]],
                             rows_v.at[b, j], sem)
            for j in range(_KB)
        ]
        for cp in cps:
            cp.wait()
        wr[b] = pltpu.async_copy(
            rows_v.at[b], out_hbm.at[pl.ds(w * _ECW + r * _KB, _KB)], wsem)
    for x in wr:
        x.wait()


def _sc_gather(h, srcs):
    k = functools.partial(
        pl.kernel,
        out_type=jax.ShapeDtypeStruct((_EC, _CH, 32), _F32),
        mesh=_sc_mesh(),
        compiler_params=_SC_PARAMS,
        scratch_types=[
            pltpu.VMEM((_ECW, _CH), jnp.int32),
            pltpu.VMEM((2, _KB, _CH, 32), _F32),
            pltpu.SemaphoreType.DMA,
            pltpu.SemaphoreType.DMA,
            pltpu.VMEM_SHARED((_NP, 32), _F32),
        ],
    )(_gather_body)
    return k(h, srcs)


def _scatter_body(msg3_hbm, dsts_hbm, z32_hbm, parts_hbm,
                  idx_v, rows_v, sem, acc_sh):
    c = lax.axis_index("c")
    s = lax.axis_index("s")
    w = s * _NC + c
    rpt = _NP // _NS
    pltpu.sync_copy(z32_hbm.at[pl.ds(s * rpt, rpt)],
                    acc_sh.at[pl.ds(s * rpt, rpt)])
    pltpu.sync_copy(dsts_hbm.at[pl.ds(w * _ECW, _ECW)], idx_v)
    plsc.subcore_barrier()
    for half in range(_ECW // _KG):
        pltpu.sync_copy(msg3_hbm.at[pl.ds(w * _ECW + half * _KG, _KG)],
                        rows_v)
        cps = [
            pltpu.async_copy(rows_v.at[j],
                             acc_sh.at[idx_v.at[half * _KG + j]],
                             sem, add=True)
            for j in range(_KG)
        ]
        for cp in cps:
            cp.wait()
    plsc.subcore_barrier()
    pltpu.sync_copy(acc_sh.at[pl.ds(s * rpt, rpt)],
                    parts_hbm.at[c, pl.ds(s * rpt, rpt)])


def _sc_scatter(msg3, dsts, z32):
    k = functools.partial(
        pl.kernel,
        out_type=jax.ShapeDtypeStruct((_NC, _NP, 32), _F32),
        mesh=_sc_mesh(),
        compiler_params=_SC_PARAMS,
        scratch_types=[
            pltpu.VMEM((_ECW, _CH), jnp.int32),
            pltpu.VMEM((_KG, _CH, 32), _F32),
            pltpu.SemaphoreType.DMA,
            pltpu.VMEM_SHARED((_NP, 32), _F32),
        ],
    )(_scatter_body)
    return k(msg3, dsts, z32)


_KR = 4


def _readout_body(h_hbm, eemb3_hbm, srcs_hbm, dsts_hbm, n2g_hbm,
                  z32_hbm, z64_hbm, zg32_hbm, zg64_hbm,
                  outh_hbm, oute_hbm,
                  idxs_v, idxd_v, idxn_v, rows32_v, rows64_v, sem,
                  acc_h, acc_e, acc_gh, acc_ge):
    c = lax.axis_index("c")
    s = lax.axis_index("s")
    w = s * _NC + c
    rpt = _NP // _NS
    gpt = _G // _NS
    npt = (_NP // _CH) // _NS
    pltpu.sync_copy(z32_hbm.at[pl.ds(s * rpt, rpt)],
                    acc_h.at[pl.ds(s * rpt, rpt)])
    pltpu.sync_copy(z64_hbm.at[pl.ds(s * rpt, rpt)],
                    acc_e.at[pl.ds(s * rpt, rpt)])
    pltpu.sync_copy(zg32_hbm.at[pl.ds(s * gpt, gpt)],
                    acc_gh.at[pl.ds(s * gpt, gpt)])
    pltpu.sync_copy(zg64_hbm.at[pl.ds(s * gpt, gpt)],
                    acc_ge.at[pl.ds(s * gpt, gpt)])
    @pl.when(s == 0)
    def _():
        pltpu.sync_copy(zg32_hbm.at[pl.ds(_G, _GP - _G)],
                        acc_gh.at[pl.ds(_G, _GP - _G)])
        pltpu.sync_copy(zg64_hbm.at[pl.ds(_G, _GP - _G)],
                        acc_ge.at[pl.ds(_G, _GP - _G)])
    pltpu.sync_copy(srcs_hbm.at[pl.ds(w * _ECW, _ECW)], idxs_v)
    pltpu.sync_copy(dsts_hbm.at[pl.ds(w * _ECW, _ECW)], idxd_v)
    plsc.subcore_barrier()
    for r in range(_ECW // _KR):
        cps = [
            pltpu.async_copy(h_hbm.at[idxs_v.at[r * _KR + j]],
                             rows32_v.at[j], sem)
            for j in range(_KR)
        ]
        pltpu.sync_copy(eemb3_hbm.at[pl.ds(w * _ECW + r * _KR, _KR)],
                        rows64_v)
        for cp in cps:
            cp.wait()
        cps = [
            pltpu.async_copy(rows32_v.at[j],
                             acc_h.at[idxd_v.at[r * _KR + j]],
                             sem, add=True)
            for j in range(_KR)
        ] + [
            pltpu.async_copy(rows64_v.at[j],
                             acc_e.at[idxd_v.at[r * _KR + j]],
                             sem, add=True)
            for j in range(_KR)
        ]
        for cp in cps:
            cp.wait()
    plsc.subcore_barrier()
    pltpu.sync_copy(n2g_hbm.at[pl.ds(s * npt, npt)], idxn_v)
    for t in range(npt):
        p = s * npt + t
        pltpu.sync_copy(acc_h.at[pl.ds(p * _CH, _CH)], rows32_v.at[0])
        pltpu.sync_copy(acc_e.at[pl.ds(p * _CH, _CH)], rows64_v.at[0])
        pltpu.sync_copy(rows32_v.at[0], acc_gh.at[idxn_v.at[t]], add=True)
        pltpu.sync_copy(rows64_v.at[0], acc_ge.at[idxn_v.at[t]], add=True)
    plsc.subcore_barrier()
    pltpu.sync_copy(acc_gh.at[pl.ds(s * gpt, gpt)],
                    outh_hbm.at[c, pl.ds(s * gpt, gpt)])
    pltpu.sync_copy(acc_ge.at[pl.ds(s * gpt, gpt)],
                    oute_hbm.at[c, pl.ds(s * gpt, gpt)])


def _sc_readout(h, eemb3, srcs, dsts, n2g, z32, z64, zg32, zg64):
    k = functools.partial(
        pl.kernel,
        out_type=(
            jax.ShapeDtypeStruct((_NC, _G, 32), _F32),
            jax.ShapeDtypeStruct((_NC, _G, 64), _F32),
        ),
        mesh=_sc_mesh(),
        compiler_params=_SC_PARAMS,
        scratch_types=[
            pltpu.VMEM((_ECW, _CH), jnp.int32),
            pltpu.VMEM((_ECW, _CH), jnp.int32),
            pltpu.VMEM(((_NP // _CH) // _NS, _CH), jnp.int32),
            pltpu.VMEM((_KR, _CH, 32), _F32),
            pltpu.VMEM((_KR, _CH, 64), _F32),
            pltpu.SemaphoreType.DMA,
            pltpu.VMEM_SHARED((_NP, 32), _F32),
            pltpu.VMEM_SHARED((_NP, 64), _F32),
            pltpu.VMEM_SHARED((_GP, 32), _F32),
            pltpu.VMEM_SHARED((_GP, 64), _F32),
        ],
    )(_readout_body)
    return k(h, eemb3, srcs, dsts, n2g, z32, z64, zg32, zg64)


def kernel(node_feats, edge_feats, edge_index, node2graph, Wp, bp, We1, be1,
           We2, be2, bconv, Wih, Whh, bih, bhh, Wpe, bpe, W1, b1, g1, beta1,
           W2, b2, g2, beta2, W3, b3):
    i32 = jnp.int32
    src = edge_index[0]
    dst = edge_index[1]
    srcs = jnp.concatenate(
        [src, jnp.zeros((_EP - _E,), i32)]).reshape(_EC, _CH)
    dsts = jnp.concatenate(
        [dst, jnp.full((_EP - _E,), _N, i32)]).reshape(_EC, _CH)
    n2g = jnp.concatenate(
        [node2graph, jnp.full((_NP - _N,), _G, i32)]).reshape(_NP // _CH, _CH)
    nf_p = jnp.pad(node_feats, ((0, _NP - _N), (0, 2)))
    ef_p = jnp.pad(edge_feats, ((0, _EP - _E), (0, 2)))
    eft_p = jnp.pad(edge_feats.T, ((0, 2), (0, _EP - _E)))
    wp_p = jnp.pad(Wp, ((0, 2), (0, 0)))
    w1t_p = jnp.pad(We1.T, ((0, 0), (0, 2)))
    wpe_p = jnp.pad(Wpe, ((0, 2), (0, 0)))
    w2t = We2.reshape(64, 32, 32).reshape(64 * 32, 32).T
    be2t = be2.reshape(32, 32).T
    wiht = Wih.T
    whht = Whh.T
    z32 = jnp.zeros((_NP, 32), _F32)
    z64 = jnp.zeros((_NP, 64), _F32)
    zg32 = jnp.zeros((_GP, 32), _F32)
    zg64 = jnp.zeros((_GP, 64), _F32)

    h = _proj_h0(nf_p, wp_p, bp.reshape(1, 32))
    eht, eemb = _edge_pre(eft_p, ef_p, w1t_p, be1.reshape(64, 1),
                          wpe_p, bpe.reshape(1, 64))
    hidden = h
    for _ in range(3):
        hs3 = _sc_gather(h, srcs)
        msg = _msg(eht, hs3.reshape(_EP, 32), w2t, be2t)
        parts = _sc_scatter(msg.reshape(_EC, _CH, 32), dsts, z32)
        h, hidden = _gru(parts, h, hidden, wiht, whht,
                         bih.reshape(1, 96), bhh.reshape(1, 96),
                         bconv.reshape(1, 32))
    ph, pe = _sc_readout(h, eemb.reshape(_EC, _CH, 64), srcs, dsts, n2g,
                         z32, z64, zg32, zg64)
    return _head(ph, pe, W1, b1.reshape(1, 300), g1.reshape(1, 300),
                 beta1.reshape(1, 300), W2, b2.reshape(1, 256),
                 g2.reshape(1, 256), beta2.reshape(1, 256),
                 W3, b3.reshape(1, 138))
